# baseline (speedup 1.0000x reference)
import functools

import jax
import jax.numpy as jnp
from jax import lax
from jax.experimental import pallas as pl
from jax.experimental.pallas import tpu as pltpu
from jax.experimental.pallas import tpu_sc as plsc

B = 4096
L = 50
N_TOK = B * L
D = 64
IMG = 128

NC, NS = 2, 16
NW = NC * NS
TOK_PER_W = N_TOK // NW
IDX_VEC = 128
CHUNK_VECS = 5
CHUNK_ROWS = CHUNK_VECS * IDX_VEC
N_CHUNKS = TOK_PER_W // CHUNK_ROWS


@functools.partial(
    pl.kernel,
    mesh=plsc.VectorSubcoreMesh(core_axis_name="c", subcore_axis_name="s"),
    compiler_params=pltpu.CompilerParams(use_tc_tiling_on_sc=False),
    out_type=jax.ShapeDtypeStruct((N_TOK, D), jnp.float32),
    scratch_types=[
        pltpu.VMEM((TOK_PER_W,), jnp.int32),
        pltpu.VMEM((CHUNK_ROWS, D), jnp.float32),
        pltpu.SemaphoreType.DMA,
    ],
)
def _sc_gather(idx_hbm, table_hbm, out_hbm, idx_v, rows_v, sem):
    wid = lax.axis_index("s") * NC + lax.axis_index("c")
    pltpu.sync_copy(idx_hbm.at[wid], idx_v)

    def chunk_body(c, carry):
        copies = []
        for j in range(CHUNK_VECS):
            copies.append(
                pltpu.async_copy(
                    table_hbm.at[idx_v.at[pl.ds(c * CHUNK_ROWS + j * IDX_VEC, IDX_VEC)]],
                    rows_v.at[pl.ds(j * IDX_VEC, IDX_VEC)],
                    sem,
                )
            )
        for cp in copies:
            cp.wait()
        base = wid * TOK_PER_W + c * CHUNK_ROWS
        pltpu.sync_copy(rows_v, out_hbm.at[pl.ds(base, CHUNK_ROWS)])
        return carry

    lax.fori_loop(0, N_CHUNKS, chunk_body, 0)


BB = 64
N_BLK = B // BB


def _tc_blend_body(img_ref, e_ref, mask_ref, w_ref, b_ref, out_ref):
    x = img_ref[...]
    proj = jax.lax.dot_general(
        x, w_ref[...],
        dimension_numbers=(((2,), (0,)), ((), ())),
        preferred_element_type=jnp.float32,
    )
    proj = proj + b_ref[...]
    e = e_ref[...].reshape(BB, L, D)
    m = mask_ref[...].astype(jnp.float32)[:, :, None]
    out_ref[...] = e + m * (proj - e)


_tc_blend = pl.pallas_call(
    _tc_blend_body,
    grid=(N_BLK,),
    in_specs=[
        pl.BlockSpec((BB, L, IMG), lambda i: (i, 0, 0)),
        pl.BlockSpec((BB * L, D), lambda i: (i, 0)),
        pl.BlockSpec((BB, L), lambda i: (i, 0)),
        pl.BlockSpec((IMG, D), lambda i: (0, 0)),
        pl.BlockSpec((1, 1, D), lambda i: (0, 0, 0)),
    ],
    out_specs=pl.BlockSpec((BB, L, D), lambda i: (i, 0, 0)),
    out_shape=jax.ShapeDtypeStruct((B, L, D), jnp.float32),
)


def kernel(text_input_sequence, image_input_sequence, image_sequence_mask, table, W, b):
    idx = text_input_sequence.astype(jnp.int32)
    idx_eff = jnp.where(image_sequence_mask != 0, 0, idx).reshape(NW, TOK_PER_W)
    embs = _sc_gather(idx_eff, table)
    return _tc_blend(image_input_sequence, embs, image_sequence_mask, W,
                     b.reshape(1, 1, D))


# speedup vs baseline: 2.9462x; 2.9462x over previous
import functools

import jax
import jax.numpy as jnp
from jax import lax
from jax.experimental import pallas as pl
from jax.experimental.pallas import tpu as pltpu
from jax.experimental.pallas import tpu_sc as plsc

B = 4096
L = 50
N_TOK = B * L
D = 64
IMG = 128

NC, NS = 2, 16
NW = NC * NS
TOK_PER_W = N_TOK // NW
IDX_VEC = 128
CHUNK_VECS = 5
CHUNK_ROWS = CHUNK_VECS * IDX_VEC
N_CHUNKS = TOK_PER_W // CHUNK_ROWS


@functools.partial(
    pl.kernel,
    mesh=plsc.VectorSubcoreMesh(core_axis_name="c", subcore_axis_name="s"),
    compiler_params=pltpu.CompilerParams(use_tc_tiling_on_sc=False),
    out_type=jax.ShapeDtypeStruct((N_TOK, D), jnp.float32),
    scratch_types=[
        pltpu.VMEM((TOK_PER_W,), jnp.int32),
        pltpu.VMEM((CHUNK_ROWS, D), jnp.float32),
        pltpu.SemaphoreType.DMA,
    ],
)
def _sc_gather(idx_hbm, table_hbm, out_hbm, idx_v, rows_v, sem):
    wid = lax.axis_index("s") * NC + lax.axis_index("c")
    pltpu.sync_copy(idx_hbm.at[wid], idx_v)

    def chunk_body(c, carry):
        copies = []
        for j in range(CHUNK_VECS):
            copies.append(
                pltpu.async_copy(
                    table_hbm.at[idx_v.at[pl.ds(c * CHUNK_ROWS + j * IDX_VEC, IDX_VEC)]],
                    rows_v.at[pl.ds(j * IDX_VEC, IDX_VEC)],
                    sem,
                )
            )
        for cp in copies:
            cp.wait()
        base = wid * TOK_PER_W + c * CHUNK_ROWS
        pltpu.sync_copy(rows_v, out_hbm.at[pl.ds(base, CHUNK_ROWS)])
        return carry

    lax.fori_loop(0, N_CHUNKS, chunk_body, 0)


BB = 64
N_BLK = B // BB


def _tc_blend_body(img_ref, e_ref, mask_ref, w_ref, b_ref, out_ref):
    x = img_ref[...]
    proj = jax.lax.dot_general(
        x, w_ref[...],
        dimension_numbers=(((2,), (0,)), ((), ())),
        preferred_element_type=jnp.float32,
    )
    proj = proj + b_ref[...]
    e = e_ref[...].reshape(BB, L, D)
    m = mask_ref[...].astype(jnp.float32)[:, :, None]
    out_ref[...] = e + m * (proj - e)


_tc_blend = pl.pallas_call(
    _tc_blend_body,
    grid=(N_BLK,),
    in_specs=[
        pl.BlockSpec((BB, L, IMG), lambda i: (i, 0, 0)),
        pl.BlockSpec((BB * L, D), lambda i: (i, 0)),
        pl.BlockSpec((BB, L), lambda i: (i, 0)),
        pl.BlockSpec((IMG, D), lambda i: (0, 0)),
        pl.BlockSpec((1, 1, D), lambda i: (0, 0, 0)),
    ],
    out_specs=pl.BlockSpec((BB, L, D), lambda i: (i, 0, 0)),
    out_shape=jax.ShapeDtypeStruct((B, L, D), jnp.float32),
)


def kernel(text_input_sequence, image_input_sequence, image_sequence_mask, table, W, b):
    idx = text_input_sequence.astype(jnp.int32).reshape(NW, TOK_PER_W)
    embs = _sc_gather(idx, table)
    return _tc_blend(image_input_sequence, embs, image_sequence_mask, W,
                     b.reshape(1, 1, D))


# speedup vs baseline: 2.9893x; 1.0146x over previous
import functools

import jax
import jax.numpy as jnp
from jax import lax
from jax.experimental import pallas as pl
from jax.experimental.pallas import tpu as pltpu
from jax.experimental.pallas import tpu_sc as plsc

B = 4096
L = 50
N_TOK = B * L
D = 64
IMG = 128

NC, NS = 2, 16
NW = NC * NS
ROWS_PER_W = B // NW
RCH = 16
N_CHUNKS = ROWS_PER_W // RCH


@functools.partial(
    pl.kernel,
    mesh=plsc.VectorSubcoreMesh(core_axis_name="c", subcore_axis_name="s"),
    compiler_params=pltpu.CompilerParams(use_tc_tiling_on_sc=False),
    out_type=jax.ShapeDtypeStruct((B, L, D), jnp.float32),
    scratch_types=[
        pltpu.VMEM((ROWS_PER_W, L), jnp.int32),
        pltpu.VMEM((RCH, L, D), jnp.float32),
        pltpu.SemaphoreType.DMA,
    ],
)
def _sc_gather(idx_hbm, table_hbm, out_hbm, idx_v, rows_v, sem):
    wid = lax.axis_index("s") * NC + lax.axis_index("c")
    base_row = wid * ROWS_PER_W
    pltpu.sync_copy(idx_hbm.at[pl.ds(base_row, ROWS_PER_W)], idx_v)

    def chunk_body(c, carry):
        copies = []
        for r in range(RCH):
            copies.append(
                pltpu.async_copy(
                    table_hbm.at[idx_v.at[c * RCH + r]],
                    rows_v.at[r],
                    sem,
                )
            )
        for cp in copies:
            cp.wait()
        pltpu.sync_copy(rows_v, out_hbm.at[pl.ds(base_row + c * RCH, RCH)])
        return carry

    lax.fori_loop(0, N_CHUNKS, chunk_body, 0)


BB = 64
N_BLK = B // BB


def _tc_blend_body(img_ref, e_ref, mask_ref, w_ref, b_ref, out_ref):
    x = img_ref[...]
    proj = jax.lax.dot_general(
        x, w_ref[...],
        dimension_numbers=(((2,), (0,)), ((), ())),
        preferred_element_type=jnp.float32,
    )
    proj = proj + b_ref[...]
    e = e_ref[...]
    m = mask_ref[...].astype(jnp.float32)[:, :, None]
    out_ref[...] = e + m * (proj - e)


_tc_blend = pl.pallas_call(
    _tc_blend_body,
    grid=(N_BLK,),
    in_specs=[
        pl.BlockSpec((BB, L, IMG), lambda i: (i, 0, 0)),
        pl.BlockSpec((BB, L, D), lambda i: (i, 0, 0)),
        pl.BlockSpec((BB, L), lambda i: (i, 0)),
        pl.BlockSpec((IMG, D), lambda i: (0, 0)),
        pl.BlockSpec((1, 1, D), lambda i: (0, 0, 0)),
    ],
    out_specs=pl.BlockSpec((BB, L, D), lambda i: (i, 0, 0)),
    out_shape=jax.ShapeDtypeStruct((B, L, D), jnp.float32),
)


def kernel(text_input_sequence, image_input_sequence, image_sequence_mask, table, W, b):
    idx = text_input_sequence.astype(jnp.int32)
    embs = _sc_gather(idx, table)
    return _tc_blend(image_input_sequence, embs, image_sequence_mask, W,
                     b.reshape(1, 1, D))


# speedup vs baseline: 3.2872x; 1.0997x over previous
import functools

import jax
import jax.numpy as jnp
from jax import lax
from jax.experimental import pallas as pl
from jax.experimental.pallas import tpu as pltpu
from jax.experimental.pallas import tpu_sc as plsc

B = 4096
L = 50
N_TOK = B * L
D = 64
IMG = 128

NC, NS = 2, 16
NW = NC * NS
TOK_PER_W = N_TOK // NW
IDX_VEC = 128
CHUNK_VECS = 5
CHUNK_ROWS = CHUNK_VECS * IDX_VEC
N_CHUNKS = TOK_PER_W // CHUNK_ROWS

BPW = B // NW


@functools.partial(
    pl.kernel,
    mesh=plsc.VectorSubcoreMesh(core_axis_name="c", subcore_axis_name="s"),
    compiler_params=pltpu.CompilerParams(use_tc_tiling_on_sc=False),
    out_type=jax.ShapeDtypeStruct((N_TOK, D), jnp.float32),
    scratch_types=[
        pltpu.VMEM((TOK_PER_W,), jnp.int32),
        pltpu.VMEM((CHUNK_ROWS, D), jnp.float32),
        pltpu.SemaphoreType.DMA,
    ],
)
def _sc_gather(idx_hbm, table_hbm, out_hbm, idx_v, rows_v, sem):
    wid = lax.axis_index("s") * NC + lax.axis_index("c")
    pltpu.sync_copy(idx_hbm.at[wid], idx_v)

    def chunk_body(c, carry):
        copies = []
        for j in range(CHUNK_VECS):
            copies.append(
                pltpu.async_copy(
                    table_hbm.at[idx_v.at[pl.ds(c * CHUNK_ROWS + j * IDX_VEC, IDX_VEC)]],
                    rows_v.at[pl.ds(j * IDX_VEC, IDX_VEC)],
                    sem,
                )
            )
        for cp in copies:
            cp.wait()
        base = wid * TOK_PER_W + c * CHUNK_ROWS
        pltpu.sync_copy(rows_v, out_hbm.at[pl.ds(base, CHUNK_ROWS)])
        return carry

    lax.fori_loop(0, N_CHUNKS, chunk_body, 0)


BBL = 1024
N_BBL = B // BBL


def _tc_blend_body(img_ref, e_ref, mask_ref, w_ref, b_ref, out_ref):
    x = img_ref[0]
    proj = jax.lax.dot_general(
        w_ref[...], x,
        dimension_numbers=(((0,), (1,)), ((), ())),
        preferred_element_type=jnp.float32,
    )
    proj = proj + b_ref[0]
    e = e_ref[0]
    m = mask_ref[0].astype(jnp.float32)
    out_ref[0] = e + m * (proj - e)


_tc_blend = pl.pallas_call(
    _tc_blend_body,
    grid=(L, N_BBL),
    in_specs=[
        pl.BlockSpec((1, BBL, IMG), lambda l, j: (l, j, 0)),
        pl.BlockSpec((1, D, BBL), lambda l, j: (l, 0, j)),
        pl.BlockSpec((1, 1, BBL), lambda l, j: (l, 0, j)),
        pl.BlockSpec((IMG, D), lambda l, j: (0, 0)),
        pl.BlockSpec((1, D, 1), lambda l, j: (0, 0, 0)),
    ],
    out_specs=pl.BlockSpec((1, D, BBL), lambda l, j: (l, 0, j)),
    out_shape=jax.ShapeDtypeStruct((L, D, B), jnp.float32),
)


def kernel(text_input_sequence, image_input_sequence, image_sequence_mask, table, W, b):
    idxP = text_input_sequence.astype(jnp.int32).T
    idx32 = idxP.reshape(NW, TOK_PER_W)
    embs2d = _sc_gather(idx32, table)
    embsP = jnp.transpose(embs2d.reshape(L, B, D), (0, 2, 1))
    imgP = jnp.transpose(image_input_sequence, (1, 0, 2))
    maskP = image_sequence_mask.T.reshape(L, 1, B)
    outP = _tc_blend(imgP, embsP, maskP, W, b.reshape(1, D, 1))
    return jnp.transpose(outP, (2, 0, 1))


# speedup vs baseline: 3.5551x; 1.0815x over previous
import functools

import jax
import jax.numpy as jnp
from jax import lax
from jax.experimental import pallas as pl
from jax.experimental.pallas import tpu as pltpu
from jax.experimental.pallas import tpu_sc as plsc

B = 4096
L = 50
N_TOK = B * L
D = 64
IMG = 128
PK = 128

NC, NS = 2, 16
NW = NC * NS
TOK_PER_W = N_TOK // NW
IDX_VEC = 128
CHUNK_VECS = 5
CHUNK_ROWS = CHUNK_VECS * IDX_VEC
N_CHUNKS = TOK_PER_W // CHUNK_ROWS


@functools.partial(
    pl.kernel,
    mesh=plsc.VectorSubcoreMesh(core_axis_name="c", subcore_axis_name="s"),
    compiler_params=pltpu.CompilerParams(use_tc_tiling_on_sc=False),
    out_type=jax.ShapeDtypeStruct((N_TOK, PK), jnp.float32),
    scratch_types=[
        pltpu.VMEM((TOK_PER_W,), jnp.int32),
        pltpu.VMEM((CHUNK_ROWS, PK), jnp.float32),
        pltpu.SemaphoreType.DMA,
    ],
)
def _sc_gather(idx_hbm, table_hbm, out_hbm, idx_v, rows_v, sem):
    wid = lax.axis_index("s") * NC + lax.axis_index("c")
    pltpu.sync_copy(idx_hbm.at[wid], idx_v)

    def chunk_body(c, carry):
        copies = []
        for j in range(CHUNK_VECS):
            copies.append(
                pltpu.async_copy(
                    table_hbm.at[idx_v.at[pl.ds(c * CHUNK_ROWS + j * IDX_VEC, IDX_VEC)]],
                    rows_v.at[pl.ds(j * IDX_VEC, IDX_VEC)],
                    sem,
                )
            )
        for cp in copies:
            cp.wait()
        base = wid * TOK_PER_W + c * CHUNK_ROWS
        pltpu.sync_copy(rows_v, out_hbm.at[pl.ds(base, CHUNK_ROWS)])
        return carry

    lax.fori_loop(0, N_CHUNKS, chunk_body, 0)


BBL = 1024
N_BBL = B // BBL


def _tc_blend_body(img_ref, e_ref, mask_ref, par_ref, w_ref, b_ref, out_ref):
    x = img_ref[0]
    proj = jax.lax.dot_general(
        w_ref[...], x,
        dimension_numbers=(((0,), (1,)), ((), ())),
        preferred_element_type=jnp.float32,
    )
    proj = proj + b_ref[0]
    eT = e_ref[0].T
    el = eT[:D]
    er = eT[D:]
    p = par_ref[0].astype(jnp.float32)
    m = mask_ref[0].astype(jnp.float32)
    e = el + p * (er - el)
    out_ref[0] = e + m * (proj - e)


_tc_blend = pl.pallas_call(
    _tc_blend_body,
    grid=(L, N_BBL),
    in_specs=[
        pl.BlockSpec((1, BBL, IMG), lambda l, j: (l, j, 0)),
        pl.BlockSpec((1, BBL, PK), lambda l, j: (l, j, 0)),
        pl.BlockSpec((1, 1, BBL), lambda l, j: (l, 0, j)),
        pl.BlockSpec((1, 1, BBL), lambda l, j: (l, 0, j)),
        pl.BlockSpec((IMG, D), lambda l, j: (0, 0)),
        pl.BlockSpec((1, D, 1), lambda l, j: (0, 0, 0)),
    ],
    out_specs=pl.BlockSpec((1, D, BBL), lambda l, j: (l, 0, j)),
    out_shape=jax.ShapeDtypeStruct((L, D, B), jnp.float32),
)


def kernel(text_input_sequence, image_input_sequence, image_sequence_mask, table, W, b):
    idxP = text_input_sequence.astype(jnp.int32).T
    idxh32 = (idxP >> 1).reshape(NW, TOK_PER_W)
    parP = (idxP & 1).reshape(L, 1, B)
    tview = table.reshape(table.shape[0] // 2, PK)
    embs2 = _sc_gather(idxh32, tview)
    e3 = embs2.reshape(L, B, PK)
    imgP = jnp.transpose(image_input_sequence, (1, 0, 2))
    maskP = image_sequence_mask.T.reshape(L, 1, B)
    outP = _tc_blend(imgP, e3, maskP, parP, W, b.reshape(1, D, 1))
    return jnp.transpose(outP, (2, 0, 1))
